# baseline (speedup 1.0000x reference)
import functools

import jax
import jax.numpy as jnp
from jax.experimental import pallas as pl
from jax.experimental.pallas import tpu as pltpu

F32 = jnp.float32
BF16 = jnp.bfloat16

EB = 512
NBK = 1024


def _cdiv(a, b):
    return (a + b - 1) // b


def _lrelu(t):
    return jnp.where(t > 0, t, 0.01 * t)


def _edge_embed_body(ea_ref, w_ref, b_ref, out_ref):
    acc = jnp.dot(ea_ref[...].astype(BF16), w_ref[...], preferred_element_type=F32)
    out_ref[...] = _lrelu(acc + b_ref[...])


def _node_embed_body(x_ref, batch_ref, wn_ref, bn_ref, prw_ref, pcw_ref,
                     h_ref, pr_ref, pc_ref, gcnt_ref):
    i = pl.program_id(0)
    h = _lrelu(jnp.dot(x_ref[...].astype(BF16), wn_ref[...],
                       preferred_element_type=F32) + bn_ref[...])
    h_ref[...] = h
    hb = h.astype(BF16)
    pr_ref[...] = jnp.dot(hb, prw_ref[...], preferred_element_type=F32).astype(BF16)
    pc_ref[...] = jnp.dot(hb, pcw_ref[...], preferred_element_type=F32).astype(BF16)
    b = batch_ref[0, 0, :]
    ng = gcnt_ref.shape[0]
    ohT = (jax.lax.broadcasted_iota(jnp.int32, (ng, b.shape[0]), 0)
           == b[None, :]).astype(F32)
    cnt = jnp.sum(ohT, axis=1, keepdims=True)

    @pl.when(i == 0)
    def _():
        gcnt_ref[...] = jnp.zeros_like(gcnt_ref)

    gcnt_ref[...] += jnp.broadcast_to(cnt, gcnt_ref.shape)


def _edge_body(e_ref, g1_ref, g2_ref, eb_ref, gvec_ref, w1c_ref, b1_ref,
               w2_ref, b2_ref, w11b_ref, b11_ref, eout_ref, m1_ref):
    e = e_ref[...]
    hdim = w1c_ref.shape[1]
    acc = jnp.dot(e.astype(BF16), w1c_ref[...], preferred_element_type=F32)
    g1 = g1_ref[...]
    acc = acc + g1[:, :hdim].astype(F32) + g2_ref[...].astype(F32)
    eb = eb_ref[0, 0, :]
    ng = gvec_ref.shape[0]
    oh = (eb[:, None] == jax.lax.broadcasted_iota(
        jnp.int32, (eb.shape[0], ng), 1)).astype(BF16)
    acc = acc + jnp.dot(oh, gvec_ref[...], preferred_element_type=F32)
    hidden = jnp.maximum(acc + b1_ref[...], 0.0)
    e_new = jnp.dot(hidden.astype(BF16), w2_ref[...],
                    preferred_element_type=F32) + b2_ref[...]
    eout_ref[...] = e + e_new
    m1 = jnp.dot(e_new.astype(BF16), w11b_ref[...],
                 preferred_element_type=F32) + g1[:, hdim:].astype(F32) + b11_ref[...]
    m1_ref[...] = jnp.maximum(m1, 0.0)


def _node_body(h_ref, seg_ref, deg_ref, batch_ref, gb2_ref, w12_ref, b12_ref,
               w21a_ref, w21b_ref, b21_ref, w22_ref, b22_ref, prw_ref, pcw_ref,
               hout_ref, pr_ref, pc_ref, gsum_ref):
    i = pl.program_id(0)
    h = h_ref[...]
    deg = deg_ref[...]
    inv = 1.0 / jnp.maximum(deg, 1.0)
    nz = (deg > 0).astype(F32)
    aggbase = seg_ref[...] * inv
    aggm = jnp.dot(aggbase.astype(BF16), w12_ref[...],
                   preferred_element_type=F32) + b12_ref[...] * nz
    b = batch_ref[0, 0, :]
    ng = gb2_ref.shape[0]
    oh = (b[:, None] == jax.lax.broadcasted_iota(
        jnp.int32, (b.shape[0], ng), 1)).astype(BF16)
    t = jnp.dot(h.astype(BF16), w21a_ref[...], preferred_element_type=F32)
    t = t + jnp.dot(aggm.astype(BF16), w21b_ref[...], preferred_element_type=F32)
    t = t + jnp.dot(oh, gb2_ref[...], preferred_element_type=F32)
    t = jnp.maximum(t + b21_ref[...], 0.0)
    h_new = jnp.dot(t.astype(BF16), w22_ref[...],
                    preferred_element_type=F32) + b22_ref[...]
    hout = h + h_new
    hout_ref[...] = hout
    hb = hout.astype(BF16)
    pr_ref[...] = jnp.dot(hb, prw_ref[...], preferred_element_type=F32).astype(BF16)
    pc_ref[...] = jnp.dot(hb, pcw_ref[...], preferred_element_type=F32).astype(BF16)
    ohT = (jax.lax.broadcasted_iota(jnp.int32, (ng, b.shape[0]), 0)
           == b[None, :]).astype(BF16)
    gs = jnp.dot(ohT, h_new.astype(BF16), preferred_element_type=F32)

    @pl.when(i == 0)
    def _():
        gsum_ref[...] = jnp.zeros_like(gsum_ref)

    gsum_ref[...] += gs


def _glob_init_body(u_ref, wg_ref, bg_ref, w1d_ref, w21c_ref,
                    g_ref, gvec_ref, gb2_ref):
    g = _lrelu(jnp.dot(u_ref[...], wg_ref[...],
                       preferred_element_type=F32) + bg_ref[...])
    g_ref[...] = g
    gvec_ref[...] = jnp.dot(g, w1d_ref[...], preferred_element_type=F32).astype(BF16)
    gb2_ref[...] = jnp.dot(g, w21c_ref[...], preferred_element_type=F32).astype(BF16)


def _glob_body(g_ref, gsum_ref, gcnt_ref, wg1a_ref, wg1b_ref, bg1_ref,
               wg2_ref, bg2_ref, w1d_ref, w21c_ref, v1w_ref, v1b_ref,
               v2w_ref, v2b_ref, gout_ref, gvec_ref, gb2_ref, val_ref):
    g = g_ref[...]
    cnt = jnp.maximum(gcnt_ref[:, :1], 1.0)
    hmean = gsum_ref[...] / cnt
    mid = jnp.maximum(
        jnp.dot(g, wg1a_ref[...], preferred_element_type=F32)
        + jnp.dot(hmean, wg1b_ref[...], preferred_element_type=F32)
        + bg1_ref[...], 0.0)
    g_new = jnp.dot(mid, wg2_ref[...], preferred_element_type=F32) + bg2_ref[...]
    gout = g + g_new
    gout_ref[...] = gout
    gvec_ref[...] = jnp.dot(gout, w1d_ref[...],
                            preferred_element_type=F32).astype(BF16)
    gb2_ref[...] = jnp.dot(gout, w21c_ref[...],
                           preferred_element_type=F32).astype(BF16)
    v = _lrelu(jnp.dot(gout, v1w_ref[...], preferred_element_type=F32) + v1b_ref[...])
    val_ref[...] = jnp.dot(v, v2w_ref[...], preferred_element_type=F32) + v2b_ref[...]


def _decode_body(h_ref, dw_ref, db_ref, out_ref):
    out_ref[...] = jnp.dot(h_ref[...], dw_ref[...],
                           preferred_element_type=F32) + db_ref[...]


def _full(shape):
    n = len(shape)
    return pl.BlockSpec(shape, lambda i: (0,) * n)


def kernel(x, edge_index, edge_attr, u, batch, params):
    N = x.shape[0]
    E = edge_index.shape[1]
    G = u.shape[0]
    D = params["node_emb"]["w"].shape[1]
    H = params["passes"][0]["edge_mlp1"]["w"].shape[1]
    VD = params["value1"]["w"].shape[1]

    EP = _cdiv(E, EB) * EB
    NP = _cdiv(N, NBK) * NBK
    if NP == N and EP > E:
        NP += NBK
    NEB = EP // EB
    NNB = NP // NBK

    row = edge_index[0].astype(jnp.int32)
    col = edge_index[1].astype(jnp.int32)
    row_p = jnp.concatenate([row, jnp.zeros((EP - E,), jnp.int32)])
    col_p = jnp.concatenate([col, jnp.full((EP - E,), NP - 1, jnp.int32)])
    batch_i = batch.astype(jnp.int32)
    batch_p = jnp.concatenate([batch_i, jnp.full((NP - N,), G, jnp.int32)])
    batch3 = batch_p.reshape(NNB, 1, NBK)
    ebatch_p = jnp.take(batch_i, row_p, axis=0)
    ebatch3 = ebatch_p.reshape(NEB, 1, EB)

    x_p = jnp.pad(x, ((0, NP - N), (0, 0)))
    ea_p = jnp.pad(edge_attr, ((0, EP - E), (0, 0)))

    def wb(lin, bf=True):
        w = lin["w"]
        return w.astype(BF16) if bf else w

    def bias(lin):
        return lin["b"][None, :]

    packs = []
    for p in params["passes"]:
        W1 = p["edge_mlp1"]["w"]
        W11 = p["node_mlp11"]["w"]
        W21 = p["node_mlp21"]["w"]
        Wg1 = p["glob_mlp1"]["w"]
        packs.append(dict(
            prw=jnp.concatenate([W1[:D], W11[:D]], axis=1).astype(BF16),
            pcw=W1[D:2 * D].astype(BF16),
            w1c=W1[2 * D:3 * D].astype(BF16),
            w1d=W1[3 * D:],
            b1=bias(p["edge_mlp1"]),
            w2=wb(p["edge_mlp2"]), b2=bias(p["edge_mlp2"]),
            w11b=W11[D:].astype(BF16), b11=bias(p["node_mlp11"]),
            w12=wb(p["node_mlp12"]), b12=bias(p["node_mlp12"]),
            w21a=W21[:D].astype(BF16), w21b=W21[D:D + H].astype(BF16),
            w21c=W21[D + H:],
            b21=bias(p["node_mlp21"]),
            w22=wb(p["node_mlp22"]), b22=bias(p["node_mlp22"]),
            wg1a=Wg1[:D], wg1b=Wg1[D:], bg1=bias(p["glob_mlp1"]),
            wg2=p["glob_mlp2"]["w"], bg2=bias(p["glob_mlp2"]),
        ))

    egrid = (NEB,)
    ngrid = (NNB,)

    e = pl.pallas_call(
        _edge_embed_body,
        grid=egrid,
        in_specs=[pl.BlockSpec((EB, edge_attr.shape[1]), lambda i: (i, 0)),
                  _full((edge_attr.shape[1], D)), _full((1, D))],
        out_specs=pl.BlockSpec((EB, D), lambda i: (i, 0)),
        out_shape=jax.ShapeDtypeStruct((EP, D), F32),
    )(ea_p, params["edge_emb"]["w"].astype(BF16), bias(params["edge_emb"]))

    h, pr, pc, gcnt = pl.pallas_call(
        _node_embed_body,
        grid=ngrid,
        in_specs=[pl.BlockSpec((NBK, x.shape[1]), lambda i: (i, 0)),
                  pl.BlockSpec((1, 1, NBK), lambda i: (i, 0, 0)),
                  _full((x.shape[1], D)), _full((1, D)),
                  _full((D, 2 * H)), _full((D, H))],
        out_specs=[pl.BlockSpec((NBK, D), lambda i: (i, 0)),
                   pl.BlockSpec((NBK, 2 * H), lambda i: (i, 0)),
                   pl.BlockSpec((NBK, H), lambda i: (i, 0)),
                   pl.BlockSpec((G, 128), lambda i: (0, 0))],
        out_shape=[jax.ShapeDtypeStruct((NP, D), F32),
                   jax.ShapeDtypeStruct((NP, 2 * H), BF16),
                   jax.ShapeDtypeStruct((NP, H), BF16),
                   jax.ShapeDtypeStruct((G, 128), F32)],
    )(x_p, batch3, params["node_emb"]["w"].astype(BF16),
      bias(params["node_emb"]), packs[0]["prw"], packs[0]["pcw"])

    g, gvec, gb2 = pl.pallas_call(
        _glob_init_body,
        grid=(1,),
        in_specs=[_full((G, u.shape[1])), _full((u.shape[1], D)), _full((1, D)),
                  _full((D, H)), _full((D, H))],
        out_specs=[_full((G, D)), _full((G, H)), _full((G, H))],
        out_shape=[jax.ShapeDtypeStruct((G, D), F32),
                   jax.ShapeDtypeStruct((G, H), BF16),
                   jax.ShapeDtypeStruct((G, H), BF16)],
    )(u, params["glob_emb"]["w"], bias(params["glob_emb"]),
      packs[0]["w1d"], packs[0]["w21c"])

    deg = jax.ops.segment_sum(jnp.ones((EP, 1), F32), col_p,
                              num_segments=NP)

    npass = len(params["passes"])
    for pi, pk in enumerate(packs):
        nxt = packs[pi + 1] if pi + 1 < npass else packs[0]

        G1 = jnp.take(pr, row_p, axis=0)
        G2 = jnp.take(pc, col_p, axis=0)

        e, m1 = pl.pallas_call(
            _edge_body,
            grid=egrid,
            in_specs=[pl.BlockSpec((EB, D), lambda i: (i, 0)),
                      pl.BlockSpec((EB, 2 * H), lambda i: (i, 0)),
                      pl.BlockSpec((EB, H), lambda i: (i, 0)),
                      pl.BlockSpec((1, 1, EB), lambda i: (i, 0, 0)),
                      _full((G, H)), _full((D, H)), _full((1, H)),
                      _full((H, D)), _full((1, D)),
                      _full((D, H)), _full((1, H))],
            out_specs=[pl.BlockSpec((EB, D), lambda i: (i, 0)),
                       pl.BlockSpec((EB, H), lambda i: (i, 0))],
            out_shape=[jax.ShapeDtypeStruct((EP, D), F32),
                       jax.ShapeDtypeStruct((EP, H), F32)],
        )(e, G1, G2, ebatch3, gvec, pk["w1c"], pk["b1"], pk["w2"], pk["b2"],
          pk["w11b"], pk["b11"])

        seg = jax.ops.segment_sum(m1, col_p, num_segments=NP)

        h, pr, pc, gsum = pl.pallas_call(
            _node_body,
            grid=ngrid,
            in_specs=[pl.BlockSpec((NBK, D), lambda i: (i, 0)),
                      pl.BlockSpec((NBK, H), lambda i: (i, 0)),
                      pl.BlockSpec((NBK, 1), lambda i: (i, 0)),
                      pl.BlockSpec((1, 1, NBK), lambda i: (i, 0, 0)),
                      _full((G, H)), _full((H, H)), _full((1, H)),
                      _full((D, H)), _full((H, H)), _full((1, H)),
                      _full((H, D)), _full((1, D)),
                      _full((D, 2 * H)), _full((D, H))],
            out_specs=[pl.BlockSpec((NBK, D), lambda i: (i, 0)),
                       pl.BlockSpec((NBK, 2 * H), lambda i: (i, 0)),
                       pl.BlockSpec((NBK, H), lambda i: (i, 0)),
                       pl.BlockSpec((G, D), lambda i: (0, 0))],
            out_shape=[jax.ShapeDtypeStruct((NP, D), F32),
                       jax.ShapeDtypeStruct((NP, 2 * H), BF16),
                       jax.ShapeDtypeStruct((NP, H), BF16),
                       jax.ShapeDtypeStruct((G, D), F32)],
        )(h, seg, deg, batch3, gb2, pk["w12"], pk["b12"], pk["w21a"],
          pk["w21b"], pk["b21"], pk["w22"], pk["b22"], nxt["prw"], nxt["pcw"])

        g, gvec, gb2, val = pl.pallas_call(
            _glob_body,
            grid=(1,),
            in_specs=[_full((G, D)), _full((G, D)), _full((G, 128)),
                      _full((D, H)), _full((D, H)), _full((1, H)),
                      _full((H, D)), _full((1, D)),
                      _full((D, H)), _full((D, H)),
                      _full((D, VD)), _full((1, VD)),
                      _full((VD, 128)), _full((1, 128))],
            out_specs=[_full((G, D)), _full((G, H)), _full((G, H)),
                       _full((G, 128))],
            out_shape=[jax.ShapeDtypeStruct((G, D), F32),
                       jax.ShapeDtypeStruct((G, H), BF16),
                       jax.ShapeDtypeStruct((G, H), BF16),
                       jax.ShapeDtypeStruct((G, 128), F32)],
        )(g, gsum, gcnt, pk["wg1a"], pk["wg1b"], pk["bg1"], pk["wg2"],
          pk["bg2"], nxt["w1d"], nxt["w21c"], params["value1"]["w"],
          bias(params["value1"]),
          jnp.pad(params["value2"]["w"], ((0, 0), (0, 127))),
          jnp.pad(bias(params["value2"]), ((0, 0), (0, 127))))

    x_out = pl.pallas_call(
        _decode_body,
        grid=ngrid,
        in_specs=[pl.BlockSpec((NBK, D), lambda i: (i, 0)),
                  _full((D, 128)), _full((1, 128))],
        out_specs=pl.BlockSpec((NBK, 128), lambda i: (i, 0)),
        out_shape=jax.ShapeDtypeStruct((NP, 128), F32),
    )(h, jnp.pad(params["node_dec"]["w"], ((0, 0), (0, 127))),
      jnp.pad(bias(params["node_dec"]), ((0, 0), (0, 127))))

    return (x_out[:N, :1], val[:, :1])


# speedup vs baseline: 2.1444x; 2.1444x over previous
import functools

import jax
import jax.numpy as jnp
from jax import lax
from jax.experimental import pallas as pl
from jax.experimental.pallas import tpu as pltpu
from jax.experimental.pallas import tpu_sc as plsc

F32 = jnp.float32
BF16 = jnp.bfloat16
I32 = jnp.int32

EB = 512
NBK = 1024
SC_C = 128
NW = 32
NT = 16


def _cdiv(a, b):
    return (a + b - 1) // b


def _lrelu(t):
    return jnp.where(t > 0, t, 0.01 * t)


def _sc_gather(htab, row_p, col_p):
    NPn, Dn = htab.shape
    EP = row_p.shape[0]
    EW = EP // NW
    nchunk = EW // SC_C
    mesh = plsc.VectorSubcoreMesh(core_axis_name="c", subcore_axis_name="s")

    @functools.partial(
        pl.kernel,
        out_type=[jax.ShapeDtypeStruct((EP, Dn), F32),
                  jax.ShapeDtypeStruct((EP, Dn), F32)],
        mesh=mesh,
        scratch_types=[pltpu.VMEM((SC_C,), I32), pltpu.VMEM((SC_C,), I32),
                       pltpu.VMEM((SC_C, Dn), F32),
                       pltpu.VMEM((SC_C, Dn), F32),
                       pltpu.SemaphoreType.DMA, pltpu.SemaphoreType.DMA],
    )
    def k(hb_hbm, row_hbm, col_hbm, hr_hbm, hc_hbm,
          idx_r, idx_c, buf_r, buf_c, sem_r, sem_c):
        wid = lax.axis_index("s") * 2 + lax.axis_index("c")
        base = wid * EW

        @pl.loop(0, nchunk)
        def _(ci):
            off = base + ci * SC_C
            pltpu.sync_copy(row_hbm.at[pl.ds(off, SC_C)], idx_r)
            pltpu.sync_copy(col_hbm.at[pl.ds(off, SC_C)], idx_c)
            cp_r = pltpu.async_copy(hb_hbm.at[idx_r], buf_r, sem_r)
            cp_c = pltpu.async_copy(hb_hbm.at[idx_c], buf_c, sem_c)
            cp_r.wait()
            cp_c.wait()
            pltpu.sync_copy(buf_r, hr_hbm.at[pl.ds(off, SC_C)])
            pltpu.sync_copy(buf_c, hc_hbm.at[pl.ds(off, SC_C)])

    return k(htab, row_p, col_p)


def _sc_scatter(m1, col_p, NPn):
    EP, H = m1.shape
    nch = H // 128
    cpc = nch // 2
    EW = EP // NT
    nchunk = EW // SC_C
    RPT = NPn // NT
    nz = RPT // SC_C
    mesh = plsc.VectorSubcoreMesh(core_axis_name="c", subcore_axis_name="s")

    @functools.partial(
        pl.kernel,
        out_type=jax.ShapeDtypeStruct((NPn, H), F32),
        mesh=mesh,
        scratch_types=[pltpu.VMEM_SHARED((NPn, 128), F32),
                       pltpu.VMEM((SC_C, 128), F32),
                       pltpu.VMEM((SC_C,), I32),
                       pltpu.VMEM((SC_C, 128), F32),
                       pltpu.SemaphoreType.DMA],
    )
    def k(m1_hbm, col_hbm, seg_hbm, acc, zbuf, idx, buf, sem):
        cid = lax.axis_index("c")
        sid = lax.axis_index("s")

        @pl.loop(0, SC_C)
        def _(r):
            @pl.loop(0, 8)
            def _(cc):
                zbuf[r, pl.ds(cc * 16, 16)] = jnp.zeros((16,), F32)

        for kk in range(cpc):
            coff = (cid * cpc + kk) * 128

            @pl.loop(0, nz)
            def _(zi):
                pltpu.sync_copy(zbuf, acc.at[pl.ds(sid * RPT + zi * SC_C, SC_C)])

            plsc.subcore_barrier()

            @pl.loop(0, nchunk)
            def _(ci):
                off = sid * EW + ci * SC_C
                pltpu.sync_copy(col_hbm.at[pl.ds(off, SC_C)], idx)
                pltpu.async_copy(
                    m1_hbm.at[pl.ds(off, SC_C), pl.ds(coff, 128)], buf, sem
                ).wait()
                pltpu.sync_copy(buf, acc.at[idx], add=True)

            plsc.subcore_barrier()
            pltpu.sync_copy(
                acc.at[pl.ds(sid * RPT, RPT)],
                seg_hbm.at[pl.ds(sid * RPT, RPT), pl.ds(coff, 128)])
            plsc.subcore_barrier()

    return k(m1, col_p)


def _sc_deg(col_p, NPn):
    EP = col_p.shape[0]
    EW = EP // NW
    nchunk = EW // SC_C
    RPT = NPn // NT
    mesh = plsc.VectorSubcoreMesh(core_axis_name="c", subcore_axis_name="s")

    @functools.partial(
        pl.kernel,
        out_type=jax.ShapeDtypeStruct((2, NPn, 128), F32),
        mesh=mesh,
        scratch_types=[pltpu.VMEM_SHARED((NPn, 128), F32),
                       pltpu.VMEM((SC_C, 128), F32),
                       pltpu.VMEM((SC_C, 128), F32),
                       pltpu.VMEM((SC_C,), I32)],
    )
    def k(col_hbm, deg_hbm, acc, ones, zbuf, idx):
        cid = lax.axis_index("c")
        sid = lax.axis_index("s")

        @pl.loop(0, SC_C)
        def _(r):
            @pl.loop(0, 8)
            def _(cc):
                ones[r, pl.ds(cc * 16, 16)] = jnp.full((16,), 1.0, F32)
                zbuf[r, pl.ds(cc * 16, 16)] = jnp.zeros((16,), F32)

        @pl.loop(0, RPT // SC_C)
        def _(zi):
            pltpu.sync_copy(zbuf, acc.at[pl.ds(sid * RPT + zi * SC_C, SC_C)])

        plsc.subcore_barrier()

        @pl.loop(0, nchunk)
        def _(ci):
            off = (cid * NT + sid) * EW + ci * SC_C
            pltpu.sync_copy(col_hbm.at[pl.ds(off, SC_C)], idx)
            pltpu.sync_copy(ones, acc.at[idx], add=True)

        plsc.subcore_barrier()
        pltpu.sync_copy(acc.at[pl.ds(sid * RPT, RPT)],
                        deg_hbm.at[cid, pl.ds(sid * RPT, RPT)])

    return k(col_p)


def _edge_embed_body(ea_ref, w_ref, b_ref, out_ref):
    acc = jnp.dot(ea_ref[...].astype(BF16), w_ref[...], preferred_element_type=F32)
    out_ref[...] = _lrelu(acc + b_ref[...])


def _node_embed_body(x_ref, batch_ref, wn_ref, bn_ref,
                     h_ref, gcnt_ref, gcr_ref):
    i = pl.program_id(0)
    h = _lrelu(jnp.dot(x_ref[...].astype(BF16), wn_ref[...],
                       preferred_element_type=F32) + bn_ref[...])
    h_ref[...] = h
    b = batch_ref[0, 0, :]
    ng = gcnt_ref.shape[0]
    ohT = (jax.lax.broadcasted_iota(I32, (ng, b.shape[0]), 0)
           == b[None, :]).astype(F32)
    cnt = jnp.sum(ohT, axis=1, keepdims=True)
    ohb = (b[:, None] == jax.lax.broadcasted_iota(
        I32, (b.shape[0], ng), 1)).astype(BF16)
    gcr = jnp.dot(jnp.ones((1, b.shape[0]), BF16), ohb,
                  preferred_element_type=F32)

    @pl.when(i == 0)
    def _():
        gcnt_ref[...] = jnp.zeros_like(gcnt_ref)
        gcr_ref[...] = jnp.zeros_like(gcr_ref)

    gcnt_ref[...] += jnp.broadcast_to(cnt, gcnt_ref.shape)
    gcr_ref[...] += gcr


def _edge_body(e_ref, hr_ref, hc_ref, row_ref, gcr_ref, gvec_ref,
               w1a_ref, w1b_ref, w1c_ref, b1_ref, w2_ref, b2_ref,
               w11a_ref, w11b_ref, b11_ref, eout_ref, m1_ref):
    e = e_ref[...]
    hr = hr_ref[...].astype(BF16)
    hc = hc_ref[...].astype(BF16)
    acc = jnp.dot(hr, w1a_ref[...], preferred_element_type=F32)
    acc = acc + jnp.dot(hc, w1b_ref[...], preferred_element_type=F32)
    acc = acc + jnp.dot(e.astype(BF16), w1c_ref[...], preferred_element_type=F32)
    ng = gvec_ref.shape[0]
    gcr = gcr_ref[...]
    tri = (jax.lax.broadcasted_iota(I32, (ng, ng), 0)
           < jax.lax.broadcasted_iota(I32, (ng, ng), 1)).astype(F32)
    starts = jnp.dot(gcr, tri, preferred_element_type=F32)
    ends = starts + gcr
    rowf = row_ref[0, 0, :].astype(F32)[:, None]
    oh = ((rowf >= starts) & (rowf < ends)).astype(BF16)
    acc = acc + jnp.dot(oh, gvec_ref[...], preferred_element_type=F32)
    hidden = jnp.maximum(acc + b1_ref[...], 0.0)
    e_new = jnp.dot(hidden.astype(BF16), w2_ref[...],
                    preferred_element_type=F32) + b2_ref[...]
    eout_ref[...] = e + e_new
    m1 = jnp.dot(e_new.astype(BF16), w11b_ref[...], preferred_element_type=F32)
    m1 = m1 + jnp.dot(hr, w11a_ref[...], preferred_element_type=F32) + b11_ref[...]
    m1_ref[...] = jnp.maximum(m1, 0.0)


def _node_body(h_ref, seg_ref, deg_ref, batch_ref, gb2_ref, w12_ref, b12_ref,
               w21a_ref, w21b_ref, b21_ref, w22_ref, b22_ref,
               hout_ref, gsum_ref):
    i = pl.program_id(0)
    h = h_ref[...]
    dg = deg_ref[...]
    deg = dg[0, :, :1] + dg[1, :, :1]
    inv = 1.0 / jnp.maximum(deg, 1.0)
    nz = (deg > 0).astype(F32)
    aggbase = seg_ref[...] * inv
    aggm = jnp.dot(aggbase.astype(BF16), w12_ref[...],
                   preferred_element_type=F32) + b12_ref[...] * nz
    b = batch_ref[0, 0, :]
    ng = gb2_ref.shape[0]
    oh = (b[:, None] == jax.lax.broadcasted_iota(
        I32, (b.shape[0], ng), 1)).astype(BF16)
    t = jnp.dot(h.astype(BF16), w21a_ref[...], preferred_element_type=F32)
    t = t + jnp.dot(aggm.astype(BF16), w21b_ref[...], preferred_element_type=F32)
    t = t + jnp.dot(oh, gb2_ref[...], preferred_element_type=F32)
    t = jnp.maximum(t + b21_ref[...], 0.0)
    h_new = jnp.dot(t.astype(BF16), w22_ref[...],
                    preferred_element_type=F32) + b22_ref[...]
    hout = h + h_new
    hout_ref[...] = hout
    ohT = (jax.lax.broadcasted_iota(I32, (ng, b.shape[0]), 0)
           == b[None, :]).astype(BF16)
    gs = jnp.dot(ohT, h_new.astype(BF16), preferred_element_type=F32)

    @pl.when(i == 0)
    def _():
        gsum_ref[...] = jnp.zeros_like(gsum_ref)

    gsum_ref[...] += gs


def _glob_init_body(u_ref, wg_ref, bg_ref, w1d_ref, w21c_ref,
                    g_ref, gvec_ref, gb2_ref):
    g = _lrelu(jnp.dot(u_ref[...], wg_ref[...],
                       preferred_element_type=F32) + bg_ref[...])
    g_ref[...] = g
    gvec_ref[...] = jnp.dot(g, w1d_ref[...], preferred_element_type=F32).astype(BF16)
    gb2_ref[...] = jnp.dot(g, w21c_ref[...], preferred_element_type=F32).astype(BF16)


def _glob_body(g_ref, gsum_ref, gcnt_ref, wg1a_ref, wg1b_ref, bg1_ref,
               wg2_ref, bg2_ref, w1d_ref, w21c_ref, v1w_ref, v1b_ref,
               v2w_ref, v2b_ref, gout_ref, gvec_ref, gb2_ref, val_ref):
    g = g_ref[...]
    cnt = jnp.maximum(gcnt_ref[:, :1], 1.0)
    hmean = gsum_ref[...] / cnt
    mid = jnp.maximum(
        jnp.dot(g, wg1a_ref[...], preferred_element_type=F32)
        + jnp.dot(hmean, wg1b_ref[...], preferred_element_type=F32)
        + bg1_ref[...], 0.0)
    g_new = jnp.dot(mid, wg2_ref[...], preferred_element_type=F32) + bg2_ref[...]
    gout = g + g_new
    gout_ref[...] = gout
    gvec_ref[...] = jnp.dot(gout, w1d_ref[...],
                            preferred_element_type=F32).astype(BF16)
    gb2_ref[...] = jnp.dot(gout, w21c_ref[...],
                           preferred_element_type=F32).astype(BF16)
    v = _lrelu(jnp.dot(gout, v1w_ref[...], preferred_element_type=F32) + v1b_ref[...])
    val_ref[...] = jnp.dot(v, v2w_ref[...], preferred_element_type=F32) + v2b_ref[...]


def _decode_body(h_ref, dw_ref, db_ref, out_ref):
    out_ref[...] = jnp.dot(h_ref[...], dw_ref[...],
                           preferred_element_type=F32) + db_ref[...]


def _full(shape):
    n = len(shape)
    return pl.BlockSpec(shape, lambda i: (0,) * n)


def kernel(x, edge_index, edge_attr, u, batch, params):
    N = x.shape[0]
    E = edge_index.shape[1]
    G = u.shape[0]
    D = params["node_emb"]["w"].shape[1]
    H = params["passes"][0]["edge_mlp1"]["w"].shape[1]
    VD = params["value1"]["w"].shape[1]

    EP = _cdiv(E, NW * SC_C) * (NW * SC_C)
    NP = _cdiv(N, NBK) * NBK
    if NP == N and EP > E:
        NP += NBK
    NEB = EP // EB
    NNB = NP // NBK

    row = edge_index[0].astype(I32)
    col = edge_index[1].astype(I32)
    row_p = jnp.concatenate([row, jnp.zeros((EP - E,), I32)])
    col_p = jnp.concatenate([col, jnp.full((EP - E,), NP - 1, I32)])
    batch_i = batch.astype(I32)
    batch_p = jnp.concatenate([batch_i, jnp.full((NP - N,), G, I32)])
    batch3 = batch_p.reshape(NNB, 1, NBK)
    row3 = row_p.reshape(NEB, 1, EB)

    x_p = jnp.pad(x, ((0, NP - N), (0, 0)))
    ea_p = jnp.pad(edge_attr, ((0, EP - E), (0, 0)))

    def bias(lin):
        return lin["b"][None, :]

    packs = []
    for p in params["passes"]:
        W1 = p["edge_mlp1"]["w"]
        W11 = p["node_mlp11"]["w"]
        W21 = p["node_mlp21"]["w"]
        Wg1 = p["glob_mlp1"]["w"]
        packs.append(dict(
            w1a=W1[:D].astype(BF16), w1b=W1[D:2 * D].astype(BF16),
            w1c=W1[2 * D:3 * D].astype(BF16), w1d=W1[3 * D:],
            b1=bias(p["edge_mlp1"]),
            w2=p["edge_mlp2"]["w"].astype(BF16), b2=bias(p["edge_mlp2"]),
            w11a=W11[:D].astype(BF16), w11b=W11[D:].astype(BF16),
            b11=bias(p["node_mlp11"]),
            w12=p["node_mlp12"]["w"].astype(BF16), b12=bias(p["node_mlp12"]),
            w21a=W21[:D].astype(BF16), w21b=W21[D:D + H].astype(BF16),
            w21c=W21[D + H:], b21=bias(p["node_mlp21"]),
            w22=p["node_mlp22"]["w"].astype(BF16), b22=bias(p["node_mlp22"]),
            wg1a=Wg1[:D], wg1b=Wg1[D:], bg1=bias(p["glob_mlp1"]),
            wg2=p["glob_mlp2"]["w"], bg2=bias(p["glob_mlp2"]),
        ))

    egrid = (NEB,)
    ngrid = (NNB,)

    e = pl.pallas_call(
        _edge_embed_body,
        grid=egrid,
        in_specs=[pl.BlockSpec((EB, edge_attr.shape[1]), lambda i: (i, 0)),
                  _full((edge_attr.shape[1], D)), _full((1, D))],
        out_specs=pl.BlockSpec((EB, D), lambda i: (i, 0)),
        out_shape=jax.ShapeDtypeStruct((EP, D), F32),
    )(ea_p, params["edge_emb"]["w"].astype(BF16), bias(params["edge_emb"]))

    h, gcnt, gcr = pl.pallas_call(
        _node_embed_body,
        grid=ngrid,
        in_specs=[pl.BlockSpec((NBK, x.shape[1]), lambda i: (i, 0)),
                  pl.BlockSpec((1, 1, NBK), lambda i: (i, 0, 0)),
                  _full((x.shape[1], D)), _full((1, D))],
        out_specs=[pl.BlockSpec((NBK, D), lambda i: (i, 0)),
                   pl.BlockSpec((G, 128), lambda i: (0, 0)),
                   pl.BlockSpec((1, G), lambda i: (0, 0))],
        out_shape=[jax.ShapeDtypeStruct((NP, D), F32),
                   jax.ShapeDtypeStruct((G, 128), F32),
                   jax.ShapeDtypeStruct((1, G), F32)],
    )(x_p, batch3, params["node_emb"]["w"].astype(BF16),
      bias(params["node_emb"]))

    g, gvec, gb2 = pl.pallas_call(
        _glob_init_body,
        grid=(1,),
        in_specs=[_full((G, u.shape[1])), _full((u.shape[1], D)), _full((1, D)),
                  _full((D, H)), _full((D, H))],
        out_specs=[_full((G, D)), _full((G, H)), _full((G, H))],
        out_shape=[jax.ShapeDtypeStruct((G, D), F32),
                   jax.ShapeDtypeStruct((G, H), BF16),
                   jax.ShapeDtypeStruct((G, H), BF16)],
    )(u, params["glob_emb"]["w"], bias(params["glob_emb"]),
      packs[0]["w1d"], packs[0]["w21c"])

    deg2 = _sc_deg(col_p, NP)

    npass = len(params["passes"])
    for pi, pk in enumerate(packs):
        nxt = packs[pi + 1] if pi + 1 < npass else packs[0]

        hr, hc = _sc_gather(h, row_p, col_p)

        e, m1 = pl.pallas_call(
            _edge_body,
            grid=egrid,
            in_specs=[pl.BlockSpec((EB, D), lambda i: (i, 0)),
                      pl.BlockSpec((EB, D), lambda i: (i, 0)),
                      pl.BlockSpec((EB, D), lambda i: (i, 0)),
                      pl.BlockSpec((1, 1, EB), lambda i: (i, 0, 0)),
                      _full((1, G)), _full((G, H)),
                      _full((D, H)), _full((D, H)), _full((D, H)), _full((1, H)),
                      _full((H, D)), _full((1, D)),
                      _full((D, H)), _full((D, H)), _full((1, H))],
            out_specs=[pl.BlockSpec((EB, D), lambda i: (i, 0)),
                       pl.BlockSpec((EB, H), lambda i: (i, 0))],
            out_shape=[jax.ShapeDtypeStruct((EP, D), F32),
                       jax.ShapeDtypeStruct((EP, H), F32)],
        )(e, hr, hc, row3, gcr, gvec, pk["w1a"], pk["w1b"], pk["w1c"],
          pk["b1"], pk["w2"], pk["b2"], pk["w11a"], pk["w11b"], pk["b11"])

        seg = _sc_scatter(m1, col_p, NP)

        h, gsum = pl.pallas_call(
            _node_body,
            grid=ngrid,
            in_specs=[pl.BlockSpec((NBK, D), lambda i: (i, 0)),
                      pl.BlockSpec((NBK, H), lambda i: (i, 0)),
                      pl.BlockSpec((2, NBK, 128), lambda i: (0, i, 0)),
                      pl.BlockSpec((1, 1, NBK), lambda i: (i, 0, 0)),
                      _full((G, H)), _full((H, H)), _full((1, H)),
                      _full((D, H)), _full((H, H)), _full((1, H)),
                      _full((H, D)), _full((1, D))],
            out_specs=[pl.BlockSpec((NBK, D), lambda i: (i, 0)),
                       pl.BlockSpec((G, D), lambda i: (0, 0))],
            out_shape=[jax.ShapeDtypeStruct((NP, D), F32),
                       jax.ShapeDtypeStruct((G, D), F32)],
        )(h, seg, deg2, batch3, gb2, pk["w12"], pk["b12"], pk["w21a"],
          pk["w21b"], pk["b21"], pk["w22"], pk["b22"])

        g, gvec, gb2, val = pl.pallas_call(
            _glob_body,
            grid=(1,),
            in_specs=[_full((G, D)), _full((G, D)), _full((G, 128)),
                      _full((D, H)), _full((D, H)), _full((1, H)),
                      _full((H, D)), _full((1, D)),
                      _full((D, H)), _full((D, H)),
                      _full((D, VD)), _full((1, VD)),
                      _full((VD, 128)), _full((1, 128))],
            out_specs=[_full((G, D)), _full((G, H)), _full((G, H)),
                       _full((G, 128))],
            out_shape=[jax.ShapeDtypeStruct((G, D), F32),
                       jax.ShapeDtypeStruct((G, H), BF16),
                       jax.ShapeDtypeStruct((G, H), BF16),
                       jax.ShapeDtypeStruct((G, 128), F32)],
        )(g, gsum, gcnt, pk["wg1a"], pk["wg1b"], pk["bg1"], pk["wg2"],
          pk["bg2"], nxt["w1d"], nxt["w21c"], params["value1"]["w"],
          bias(params["value1"]),
          jnp.pad(params["value2"]["w"], ((0, 0), (0, 127))),
          jnp.pad(bias(params["value2"]), ((0, 0), (0, 127))))

    x_out = pl.pallas_call(
        _decode_body,
        grid=ngrid,
        in_specs=[pl.BlockSpec((NBK, D), lambda i: (i, 0)),
                  _full((D, 128)), _full((1, 128))],
        out_specs=pl.BlockSpec((NBK, 128), lambda i: (i, 0)),
        out_shape=jax.ShapeDtypeStruct((NP, 128), F32),
    )(h, jnp.pad(params["node_dec"]["w"], ((0, 0), (0, 127))),
      jnp.pad(bias(params["node_dec"]), ((0, 0), (0, 127))))

    return (x_out[:N, :1], val[:, :1])


# speedup vs baseline: 2.6995x; 1.2589x over previous
import functools

import jax
import jax.numpy as jnp
from jax import lax
from jax.experimental import pallas as pl
from jax.experimental.pallas import tpu as pltpu
from jax.experimental.pallas import tpu_sc as plsc

F32 = jnp.float32
BF16 = jnp.bfloat16
I32 = jnp.int32

EB = 512
NBK = 1024
SC_C = 128
NW = 32
NT = 16


def _cdiv(a, b):
    return (a + b - 1) // b


def _lrelu(t):
    return jnp.where(t > 0, t, 0.01 * t)


def _sc_gather(htab, row_p, col_p):
    NPn, Dn = htab.shape
    EP = row_p.shape[0]
    EW = EP // NW
    nchunk = EW // SC_C
    mesh = plsc.VectorSubcoreMesh(core_axis_name="c", subcore_axis_name="s")

    @functools.partial(
        pl.kernel,
        out_type=[jax.ShapeDtypeStruct((EP, Dn), F32),
                  jax.ShapeDtypeStruct((EP, Dn), F32)],
        mesh=mesh,
        scratch_types=[pltpu.VMEM((SC_C,), I32), pltpu.VMEM((SC_C,), I32),
                       pltpu.VMEM((SC_C, Dn), F32),
                       pltpu.VMEM((SC_C, Dn), F32),
                       pltpu.SemaphoreType.DMA, pltpu.SemaphoreType.DMA],
    )
    def k(hb_hbm, row_hbm, col_hbm, hr_hbm, hc_hbm,
          idx_r, idx_c, buf_r, buf_c, sem_r, sem_c):
        wid = lax.axis_index("s") * 2 + lax.axis_index("c")
        base = wid * EW

        @pl.loop(0, nchunk)
        def _(ci):
            off = base + ci * SC_C
            pltpu.sync_copy(row_hbm.at[pl.ds(off, SC_C)], idx_r)
            pltpu.sync_copy(col_hbm.at[pl.ds(off, SC_C)], idx_c)
            cp_r = pltpu.async_copy(hb_hbm.at[idx_r], buf_r, sem_r)
            cp_c = pltpu.async_copy(hb_hbm.at[idx_c], buf_c, sem_c)
            cp_r.wait()
            cp_c.wait()
            pltpu.sync_copy(buf_r, hr_hbm.at[pl.ds(off, SC_C)])
            pltpu.sync_copy(buf_c, hc_hbm.at[pl.ds(off, SC_C)])

    return k(htab, row_p, col_p)


def _sc_scatter(m1, col_p, NPn):
    EP, H = m1.shape
    nch = H // 128
    cpc = nch // 2
    EW = EP // NT
    nchunk = EW // SC_C
    RPT = NPn // NT
    nz = RPT // SC_C
    mesh = plsc.VectorSubcoreMesh(core_axis_name="c", subcore_axis_name="s")

    @functools.partial(
        pl.kernel,
        out_type=jax.ShapeDtypeStruct((NPn, H), F32),
        mesh=mesh,
        scratch_types=[pltpu.VMEM_SHARED((NPn, 128), F32),
                       pltpu.VMEM((SC_C, 128), F32),
                       pltpu.VMEM((SC_C,), I32),
                       pltpu.VMEM((SC_C, 128), F32),
                       pltpu.SemaphoreType.DMA],
    )
    def k(m1_hbm, col_hbm, seg_hbm, acc, zbuf, idx, buf, sem):
        cid = lax.axis_index("c")
        sid = lax.axis_index("s")

        @pl.loop(0, SC_C)
        def _(r):
            @pl.loop(0, 8)
            def _(cc):
                zbuf[r, pl.ds(cc * 16, 16)] = jnp.zeros((16,), F32)

        for kk in range(cpc):
            coff = (cid * cpc + kk) * 128

            @pl.loop(0, nz)
            def _(zi):
                pltpu.sync_copy(zbuf, acc.at[pl.ds(sid * RPT + zi * SC_C, SC_C)])

            plsc.subcore_barrier()

            @pl.loop(0, nchunk)
            def _(ci):
                off = sid * EW + ci * SC_C
                pltpu.sync_copy(col_hbm.at[pl.ds(off, SC_C)], idx)
                pltpu.async_copy(
                    m1_hbm.at[pl.ds(off, SC_C), pl.ds(coff, 128)], buf, sem
                ).wait()
                pltpu.sync_copy(buf, acc.at[idx], add=True)

            plsc.subcore_barrier()
            pltpu.sync_copy(
                acc.at[pl.ds(sid * RPT, RPT)],
                seg_hbm.at[pl.ds(sid * RPT, RPT), pl.ds(coff, 128)])
            plsc.subcore_barrier()

    return k(m1, col_p)


def _sc_deg(col_p, NPn):
    EP = col_p.shape[0]
    EW = EP // NW
    nchunk = EW // SC_C
    RPT = NPn // NT
    mesh = plsc.VectorSubcoreMesh(core_axis_name="c", subcore_axis_name="s")

    @functools.partial(
        pl.kernel,
        out_type=jax.ShapeDtypeStruct((2, NPn, 128), F32),
        mesh=mesh,
        scratch_types=[pltpu.VMEM_SHARED((NPn, 128), F32),
                       pltpu.VMEM((SC_C, 128), F32),
                       pltpu.VMEM((SC_C, 128), F32),
                       pltpu.VMEM((SC_C,), I32)],
    )
    def k(col_hbm, deg_hbm, acc, ones, zbuf, idx):
        cid = lax.axis_index("c")
        sid = lax.axis_index("s")

        @pl.loop(0, SC_C)
        def _(r):
            @pl.loop(0, 8)
            def _(cc):
                ones[r, pl.ds(cc * 16, 16)] = jnp.full((16,), 1.0, F32)
                zbuf[r, pl.ds(cc * 16, 16)] = jnp.zeros((16,), F32)

        @pl.loop(0, RPT // SC_C)
        def _(zi):
            pltpu.sync_copy(zbuf, acc.at[pl.ds(sid * RPT + zi * SC_C, SC_C)])

        plsc.subcore_barrier()

        @pl.loop(0, nchunk)
        def _(ci):
            off = (cid * NT + sid) * EW + ci * SC_C
            pltpu.sync_copy(col_hbm.at[pl.ds(off, SC_C)], idx)
            pltpu.sync_copy(ones, acc.at[idx], add=True)

        plsc.subcore_barrier()
        pltpu.sync_copy(acc.at[pl.ds(sid * RPT, RPT)],
                        deg_hbm.at[cid, pl.ds(sid * RPT, RPT)])

    return k(col_p)


def _edge_embed_body(ea_ref, w_ref, b_ref, out_ref):
    acc = jnp.dot(ea_ref[...].astype(BF16), w_ref[...], preferred_element_type=F32)
    out_ref[...] = _lrelu(acc + b_ref[...])


def _node_embed_body(x_ref, batch_ref, wn_ref, bn_ref,
                     h_ref, gcnt_ref, gcr_ref):
    i = pl.program_id(0)
    h = _lrelu(jnp.dot(x_ref[...].astype(BF16), wn_ref[...],
                       preferred_element_type=F32) + bn_ref[...])
    h_ref[...] = h
    b = batch_ref[0, 0, :]
    ng = gcnt_ref.shape[0]
    ohT = (jax.lax.broadcasted_iota(I32, (ng, b.shape[0]), 0)
           == b[None, :]).astype(F32)
    cnt = jnp.sum(ohT, axis=1, keepdims=True)
    ohb = (b[:, None] == jax.lax.broadcasted_iota(
        I32, (b.shape[0], ng), 1)).astype(BF16)
    gcr = jnp.dot(jnp.ones((1, b.shape[0]), BF16), ohb,
                  preferred_element_type=F32)

    @pl.when(i == 0)
    def _():
        gcnt_ref[...] = jnp.zeros_like(gcnt_ref)
        gcr_ref[...] = jnp.zeros_like(gcr_ref)

    gcnt_ref[...] += jnp.broadcast_to(cnt, gcnt_ref.shape)
    gcr_ref[...] += gcr


def _edge_body(e_ref, hr_ref, hc_ref, row_ref, gcr_ref, gvec_ref,
               w1a_ref, w1b_ref, w1c_ref, b1_ref, w2_ref, b2_ref,
               w11a_ref, w11b_ref, b11_ref, eout_ref, m1_ref):
    e = e_ref[...]
    hr = hr_ref[...].astype(BF16)
    hc = hc_ref[...].astype(BF16)
    acc = jnp.dot(hr, w1a_ref[...], preferred_element_type=F32)
    acc = acc + jnp.dot(hc, w1b_ref[...], preferred_element_type=F32)
    acc = acc + jnp.dot(e.astype(BF16), w1c_ref[...], preferred_element_type=F32)
    ng = gvec_ref.shape[0]
    gcr = gcr_ref[...]
    tri = (jax.lax.broadcasted_iota(I32, (ng, ng), 0)
           < jax.lax.broadcasted_iota(I32, (ng, ng), 1)).astype(F32)
    starts = jnp.dot(gcr, tri, preferred_element_type=F32)
    ends = starts + gcr
    rowf = row_ref[0, 0, :].astype(F32)[:, None]
    oh = ((rowf >= starts) & (rowf < ends)).astype(BF16)
    acc = acc + jnp.dot(oh, gvec_ref[...], preferred_element_type=F32)
    hidden = jnp.maximum(acc + b1_ref[...], 0.0)
    e_new = jnp.dot(hidden.astype(BF16), w2_ref[...],
                    preferred_element_type=F32) + b2_ref[...]
    eout_ref[...] = e + e_new
    m1 = jnp.dot(e_new.astype(BF16), w11b_ref[...], preferred_element_type=F32)
    m1 = m1 + jnp.dot(hr, w11a_ref[...], preferred_element_type=F32) + b11_ref[...]
    m1_ref[...] = jnp.maximum(m1, 0.0)


def _node_body(h_ref, seg0_ref, seg1_ref, deg_ref, batch_ref, gb2_ref,
               w12_ref, b12_ref,
               w21a_ref, w21b_ref, b21_ref, w22_ref, b22_ref,
               hout_ref, gsum_ref):
    i = pl.program_id(0)
    h = h_ref[...]
    dg = deg_ref[...]
    deg = dg[0, :, :1] + dg[1, :, :1]
    inv = 1.0 / jnp.maximum(deg, 1.0)
    nz = (deg > 0).astype(F32)
    aggbase = (seg0_ref[...] + seg1_ref[...]) * inv
    aggm = jnp.dot(aggbase.astype(BF16), w12_ref[...],
                   preferred_element_type=F32) + b12_ref[...] * nz
    b = batch_ref[0, 0, :]
    ng = gb2_ref.shape[0]
    oh = (b[:, None] == jax.lax.broadcasted_iota(
        I32, (b.shape[0], ng), 1)).astype(BF16)
    t = jnp.dot(h.astype(BF16), w21a_ref[...], preferred_element_type=F32)
    t = t + jnp.dot(aggm.astype(BF16), w21b_ref[...], preferred_element_type=F32)
    t = t + jnp.dot(oh, gb2_ref[...], preferred_element_type=F32)
    t = jnp.maximum(t + b21_ref[...], 0.0)
    h_new = jnp.dot(t.astype(BF16), w22_ref[...],
                    preferred_element_type=F32) + b22_ref[...]
    hout = h + h_new
    hout_ref[...] = hout
    ohT = (jax.lax.broadcasted_iota(I32, (ng, b.shape[0]), 0)
           == b[None, :]).astype(BF16)
    gs = jnp.dot(ohT, h_new.astype(BF16), preferred_element_type=F32)

    @pl.when(i == 0)
    def _():
        gsum_ref[...] = jnp.zeros_like(gsum_ref)

    gsum_ref[...] += gs


def _glob_init_body(u_ref, wg_ref, bg_ref, w1d_ref, w21c_ref,
                    g_ref, gvec_ref, gb2_ref):
    g = _lrelu(jnp.dot(u_ref[...], wg_ref[...],
                       preferred_element_type=F32) + bg_ref[...])
    g_ref[...] = g
    gvec_ref[...] = jnp.dot(g, w1d_ref[...], preferred_element_type=F32).astype(BF16)
    gb2_ref[...] = jnp.dot(g, w21c_ref[...], preferred_element_type=F32).astype(BF16)


def _glob_body(g_ref, gsum_ref, gcnt_ref, wg1a_ref, wg1b_ref, bg1_ref,
               wg2_ref, bg2_ref, w1d_ref, w21c_ref, v1w_ref, v1b_ref,
               v2w_ref, v2b_ref, gout_ref, gvec_ref, gb2_ref, val_ref):
    g = g_ref[...]
    cnt = jnp.maximum(gcnt_ref[:, :1], 1.0)
    hmean = gsum_ref[...] / cnt
    mid = jnp.maximum(
        jnp.dot(g, wg1a_ref[...], preferred_element_type=F32)
        + jnp.dot(hmean, wg1b_ref[...], preferred_element_type=F32)
        + bg1_ref[...], 0.0)
    g_new = jnp.dot(mid, wg2_ref[...], preferred_element_type=F32) + bg2_ref[...]
    gout = g + g_new
    gout_ref[...] = gout
    gvec_ref[...] = jnp.dot(gout, w1d_ref[...],
                            preferred_element_type=F32).astype(BF16)
    gb2_ref[...] = jnp.dot(gout, w21c_ref[...],
                           preferred_element_type=F32).astype(BF16)
    v = _lrelu(jnp.dot(gout, v1w_ref[...], preferred_element_type=F32) + v1b_ref[...])
    val_ref[...] = jnp.dot(v, v2w_ref[...], preferred_element_type=F32) + v2b_ref[...]


def _decode_body(h_ref, dw_ref, db_ref, out_ref):
    out_ref[...] = jnp.dot(h_ref[...], dw_ref[...],
                           preferred_element_type=F32) + db_ref[...]


def _full(shape):
    n = len(shape)
    return pl.BlockSpec(shape, lambda i: (0,) * n)


def kernel(x, edge_index, edge_attr, u, batch, params):
    N = x.shape[0]
    E = edge_index.shape[1]
    G = u.shape[0]
    D = params["node_emb"]["w"].shape[1]
    H = params["passes"][0]["edge_mlp1"]["w"].shape[1]
    VD = params["value1"]["w"].shape[1]

    EH = E // 2
    EPH = _cdiv(EH, NW * SC_C) * (NW * SC_C)
    EP = 2 * EPH
    NP = _cdiv(N, NBK) * NBK
    if NP == N and EPH > EH:
        NP += NBK
    NEB = EPH // EB
    NNB = NP // NBK

    row = edge_index[0].astype(I32)
    col = edge_index[1].astype(I32)

    def pad_half(a, lo, hi, fill):
        seg = a[lo:hi]
        return jnp.concatenate(
            [seg, jnp.full((EPH - (hi - lo),), fill, I32)])

    row_h = [pad_half(row, 0, EH, 0), pad_half(row, EH, E, 0)]
    col_h = [pad_half(col, 0, EH, NP - 1), pad_half(col, EH, E, NP - 1)]
    col_p = jnp.concatenate(col_h)
    batch_i = batch.astype(I32)
    batch_p = jnp.concatenate([batch_i, jnp.full((NP - N,), G, I32)])
    batch3 = batch_p.reshape(NNB, 1, NBK)
    row3_h = [r.reshape(NEB, 1, EB) for r in row_h]

    x_p = jnp.pad(x, ((0, NP - N), (0, 0)))
    ea_h = [jnp.pad(edge_attr[:EH], ((0, EPH - EH), (0, 0))),
            jnp.pad(edge_attr[EH:], ((0, EPH - EH), (0, 0)))]

    def bias(lin):
        return lin["b"][None, :]

    packs = []
    for p in params["passes"]:
        W1 = p["edge_mlp1"]["w"]
        W11 = p["node_mlp11"]["w"]
        W21 = p["node_mlp21"]["w"]
        Wg1 = p["glob_mlp1"]["w"]
        packs.append(dict(
            w1a=W1[:D].astype(BF16), w1b=W1[D:2 * D].astype(BF16),
            w1c=W1[2 * D:3 * D].astype(BF16), w1d=W1[3 * D:],
            b1=bias(p["edge_mlp1"]),
            w2=p["edge_mlp2"]["w"].astype(BF16), b2=bias(p["edge_mlp2"]),
            w11a=W11[:D].astype(BF16), w11b=W11[D:].astype(BF16),
            b11=bias(p["node_mlp11"]),
            w12=p["node_mlp12"]["w"].astype(BF16), b12=bias(p["node_mlp12"]),
            w21a=W21[:D].astype(BF16), w21b=W21[D:D + H].astype(BF16),
            w21c=W21[D + H:], b21=bias(p["node_mlp21"]),
            w22=p["node_mlp22"]["w"].astype(BF16), b22=bias(p["node_mlp22"]),
            wg1a=Wg1[:D], wg1b=Wg1[D:], bg1=bias(p["glob_mlp1"]),
            wg2=p["glob_mlp2"]["w"], bg2=bias(p["glob_mlp2"]),
        ))

    egrid = (NEB,)
    ngrid = (NNB,)

    e_h = [pl.pallas_call(
        _edge_embed_body,
        grid=egrid,
        in_specs=[pl.BlockSpec((EB, edge_attr.shape[1]), lambda i: (i, 0)),
                  _full((edge_attr.shape[1], D)), _full((1, D))],
        out_specs=pl.BlockSpec((EB, D), lambda i: (i, 0)),
        out_shape=jax.ShapeDtypeStruct((EPH, D), F32),
    )(ea, params["edge_emb"]["w"].astype(BF16), bias(params["edge_emb"]))
        for ea in ea_h]

    h, gcnt, gcr = pl.pallas_call(
        _node_embed_body,
        grid=ngrid,
        in_specs=[pl.BlockSpec((NBK, x.shape[1]), lambda i: (i, 0)),
                  pl.BlockSpec((1, 1, NBK), lambda i: (i, 0, 0)),
                  _full((x.shape[1], D)), _full((1, D))],
        out_specs=[pl.BlockSpec((NBK, D), lambda i: (i, 0)),
                   pl.BlockSpec((G, 128), lambda i: (0, 0)),
                   pl.BlockSpec((1, G), lambda i: (0, 0))],
        out_shape=[jax.ShapeDtypeStruct((NP, D), F32),
                   jax.ShapeDtypeStruct((G, 128), F32),
                   jax.ShapeDtypeStruct((1, G), F32)],
    )(x_p, batch3, params["node_emb"]["w"].astype(BF16),
      bias(params["node_emb"]))

    g, gvec, gb2 = pl.pallas_call(
        _glob_init_body,
        grid=(1,),
        in_specs=[_full((G, u.shape[1])), _full((u.shape[1], D)), _full((1, D)),
                  _full((D, H)), _full((D, H))],
        out_specs=[_full((G, D)), _full((G, H)), _full((G, H))],
        out_shape=[jax.ShapeDtypeStruct((G, D), F32),
                   jax.ShapeDtypeStruct((G, H), BF16),
                   jax.ShapeDtypeStruct((G, H), BF16)],
    )(u, params["glob_emb"]["w"], bias(params["glob_emb"]),
      packs[0]["w1d"], packs[0]["w21c"])

    deg2 = _sc_deg(col_p, NP)

    npass = len(params["passes"])
    for pi, pk in enumerate(packs):
        nxt = packs[pi + 1] if pi + 1 < npass else packs[0]

        seg_h = [None, None]
        for hf in range(2):
            hr, hc = _sc_gather(h, row_h[hf], col_h[hf])

            e_h[hf], m1 = pl.pallas_call(
                _edge_body,
                grid=egrid,
                in_specs=[pl.BlockSpec((EB, D), lambda i: (i, 0)),
                          pl.BlockSpec((EB, D), lambda i: (i, 0)),
                          pl.BlockSpec((EB, D), lambda i: (i, 0)),
                          pl.BlockSpec((1, 1, EB), lambda i: (i, 0, 0)),
                          _full((1, G)), _full((G, H)),
                          _full((D, H)), _full((D, H)), _full((D, H)),
                          _full((1, H)),
                          _full((H, D)), _full((1, D)),
                          _full((D, H)), _full((D, H)), _full((1, H))],
                out_specs=[pl.BlockSpec((EB, D), lambda i: (i, 0)),
                           pl.BlockSpec((EB, H), lambda i: (i, 0))],
                out_shape=[jax.ShapeDtypeStruct((EPH, D), F32),
                           jax.ShapeDtypeStruct((EPH, H), F32)],
            )(e_h[hf], hr, hc, row3_h[hf], gcr, gvec, pk["w1a"], pk["w1b"],
              pk["w1c"], pk["b1"], pk["w2"], pk["b2"], pk["w11a"],
              pk["w11b"], pk["b11"])

            seg_h[hf] = _sc_scatter(m1, col_h[hf], NP)

        h, gsum = pl.pallas_call(
            _node_body,
            grid=ngrid,
            in_specs=[pl.BlockSpec((NBK, D), lambda i: (i, 0)),
                      pl.BlockSpec((NBK, H), lambda i: (i, 0)),
                      pl.BlockSpec((NBK, H), lambda i: (i, 0)),
                      pl.BlockSpec((2, NBK, 128), lambda i: (0, i, 0)),
                      pl.BlockSpec((1, 1, NBK), lambda i: (i, 0, 0)),
                      _full((G, H)), _full((H, H)), _full((1, H)),
                      _full((D, H)), _full((H, H)), _full((1, H)),
                      _full((H, D)), _full((1, D))],
            out_specs=[pl.BlockSpec((NBK, D), lambda i: (i, 0)),
                       pl.BlockSpec((G, D), lambda i: (0, 0))],
            out_shape=[jax.ShapeDtypeStruct((NP, D), F32),
                       jax.ShapeDtypeStruct((G, D), F32)],
        )(h, seg_h[0], seg_h[1], deg2, batch3, gb2, pk["w12"], pk["b12"],
          pk["w21a"], pk["w21b"], pk["b21"], pk["w22"], pk["b22"])

        g, gvec, gb2, val = pl.pallas_call(
            _glob_body,
            grid=(1,),
            in_specs=[_full((G, D)), _full((G, D)), _full((G, 128)),
                      _full((D, H)), _full((D, H)), _full((1, H)),
                      _full((H, D)), _full((1, D)),
                      _full((D, H)), _full((D, H)),
                      _full((D, VD)), _full((1, VD)),
                      _full((VD, 128)), _full((1, 128))],
            out_specs=[_full((G, D)), _full((G, H)), _full((G, H)),
                       _full((G, 128))],
            out_shape=[jax.ShapeDtypeStruct((G, D), F32),
                       jax.ShapeDtypeStruct((G, H), BF16),
                       jax.ShapeDtypeStruct((G, H), BF16),
                       jax.ShapeDtypeStruct((G, 128), F32)],
        )(g, gsum, gcnt, pk["wg1a"], pk["wg1b"], pk["bg1"], pk["wg2"],
          pk["bg2"], nxt["w1d"], nxt["w21c"], params["value1"]["w"],
          bias(params["value1"]),
          jnp.pad(params["value2"]["w"], ((0, 0), (0, 127))),
          jnp.pad(bias(params["value2"]), ((0, 0), (0, 127))))

    x_out = pl.pallas_call(
        _decode_body,
        grid=ngrid,
        in_specs=[pl.BlockSpec((NBK, D), lambda i: (i, 0)),
                  _full((D, 128)), _full((1, 128))],
        out_specs=pl.BlockSpec((NBK, 128), lambda i: (i, 0)),
        out_shape=jax.ShapeDtypeStruct((NP, 128), F32),
    )(h, jnp.pad(params["node_dec"]["w"], ((0, 0), (0, 127))),
      jnp.pad(bias(params["node_dec"]), ((0, 0), (0, 127))))

    return (x_out[:N, :1], val[:, :1])


# speedup vs baseline: 3.0440x; 1.1276x over previous
import functools

import jax
import jax.numpy as jnp
from jax import lax
from jax.experimental import pallas as pl
from jax.experimental.pallas import tpu as pltpu
from jax.experimental.pallas import tpu_sc as plsc

F32 = jnp.float32
BF16 = jnp.bfloat16
I32 = jnp.int32

EB = 512
NBK = 1024
SC_C = 128
NW = 32
NT = 16


def _cdiv(a, b):
    return (a + b - 1) // b


def _lrelu(t):
    return jnp.where(t > 0, t, 0.01 * t)


def _pack(hf32):
    n = hf32.shape[1] // 2
    a = jax.lax.bitcast_convert_type(hf32[:, :n], jnp.uint32)
    b = jax.lax.bitcast_convert_type(hf32[:, n:], jnp.uint32)
    a16 = (a + jnp.uint32(0x8000)) >> 16
    b16 = (b + jnp.uint32(0x8000)) & jnp.uint32(0xFFFF0000)
    return jax.lax.bitcast_convert_type(a16 | b16, I32)


def _unpack_bf16(p32):
    u = jax.lax.bitcast_convert_type(p32, jnp.uint32)
    a = jax.lax.bitcast_convert_type(u << 16, F32)
    b = jax.lax.bitcast_convert_type(u & jnp.uint32(0xFFFF0000), F32)
    return jnp.concatenate([a, b], axis=1).astype(BF16)


def _sc_gather(htab, row_p, col_p):
    NPn, Dn = htab.shape
    EP = row_p.shape[0]
    EW = EP // NW
    nchunk = EW // SC_C
    mesh = plsc.VectorSubcoreMesh(core_axis_name="c", subcore_axis_name="s")

    @functools.partial(
        pl.kernel,
        out_type=[jax.ShapeDtypeStruct((EP, Dn), I32),
                  jax.ShapeDtypeStruct((EP, Dn), I32)],
        mesh=mesh,
        scratch_types=[pltpu.VMEM((SC_C,), I32), pltpu.VMEM((SC_C,), I32),
                       pltpu.VMEM((SC_C, Dn), I32),
                       pltpu.VMEM((SC_C, Dn), I32),
                       pltpu.SemaphoreType.DMA, pltpu.SemaphoreType.DMA],
    )
    def k(hb_hbm, row_hbm, col_hbm, hr_hbm, hc_hbm,
          idx_r, idx_c, buf_r, buf_c, sem_r, sem_c):
        wid = lax.axis_index("s") * 2 + lax.axis_index("c")
        base = wid * EW

        @pl.loop(0, nchunk)
        def _(ci):
            off = base + ci * SC_C
            pltpu.sync_copy(row_hbm.at[pl.ds(off, SC_C)], idx_r)
            pltpu.sync_copy(col_hbm.at[pl.ds(off, SC_C)], idx_c)
            cp_r = pltpu.async_copy(hb_hbm.at[idx_r], buf_r, sem_r)
            cp_c = pltpu.async_copy(hb_hbm.at[idx_c], buf_c, sem_c)
            cp_r.wait()
            cp_c.wait()
            pltpu.sync_copy(buf_r, hr_hbm.at[pl.ds(off, SC_C)])
            pltpu.sync_copy(buf_c, hc_hbm.at[pl.ds(off, SC_C)])

    return k(htab, row_p, col_p)


def _sc_scatter(m1, col_p, NPn):
    EP, H = m1.shape
    nch = H // 128
    cpc = nch // 2
    EW = EP // NT
    nchunk = EW // SC_C
    RPT = NPn // NT
    nz = RPT // SC_C
    mesh = plsc.VectorSubcoreMesh(core_axis_name="c", subcore_axis_name="s")

    @functools.partial(
        pl.kernel,
        out_type=jax.ShapeDtypeStruct((NPn, H), F32),
        mesh=mesh,
        scratch_types=[pltpu.VMEM_SHARED((NPn, 128), F32),
                       pltpu.VMEM((SC_C, 128), F32),
                       pltpu.VMEM((SC_C,), I32),
                       pltpu.VMEM((SC_C, 128), F32),
                       pltpu.SemaphoreType.DMA],
    )
    def k(m1_hbm, col_hbm, seg_hbm, acc, zbuf, idx, buf, sem):
        cid = lax.axis_index("c")
        sid = lax.axis_index("s")

        @pl.loop(0, SC_C)
        def _(r):
            @pl.loop(0, 8)
            def _(cc):
                zbuf[r, pl.ds(cc * 16, 16)] = jnp.zeros((16,), F32)

        for kk in range(cpc):
            coff = (cid * cpc + kk) * 128

            @pl.loop(0, nz)
            def _(zi):
                pltpu.sync_copy(zbuf, acc.at[pl.ds(sid * RPT + zi * SC_C, SC_C)])

            plsc.subcore_barrier()

            @pl.loop(0, nchunk)
            def _(ci):
                off = sid * EW + ci * SC_C
                pltpu.sync_copy(col_hbm.at[pl.ds(off, SC_C)], idx)
                pltpu.async_copy(
                    m1_hbm.at[pl.ds(off, SC_C), pl.ds(coff, 128)], buf, sem
                ).wait()
                pltpu.sync_copy(buf, acc.at[idx], add=True)

            plsc.subcore_barrier()
            pltpu.sync_copy(
                acc.at[pl.ds(sid * RPT, RPT)],
                seg_hbm.at[pl.ds(sid * RPT, RPT), pl.ds(coff, 128)])
            plsc.subcore_barrier()

    return k(m1, col_p)


def _sc_deg(col_p, NPn):
    EP = col_p.shape[0]
    EW = EP // NW
    nchunk = EW // SC_C
    RPT = NPn // NT
    mesh = plsc.VectorSubcoreMesh(core_axis_name="c", subcore_axis_name="s")

    @functools.partial(
        pl.kernel,
        out_type=jax.ShapeDtypeStruct((2, NPn, 128), F32),
        mesh=mesh,
        scratch_types=[pltpu.VMEM_SHARED((NPn, 128), F32),
                       pltpu.VMEM((SC_C, 128), F32),
                       pltpu.VMEM((SC_C, 128), F32),
                       pltpu.VMEM((SC_C,), I32)],
    )
    def k(col_hbm, deg_hbm, acc, ones, zbuf, idx):
        cid = lax.axis_index("c")
        sid = lax.axis_index("s")

        @pl.loop(0, SC_C)
        def _(r):
            @pl.loop(0, 8)
            def _(cc):
                ones[r, pl.ds(cc * 16, 16)] = jnp.full((16,), 1.0, F32)
                zbuf[r, pl.ds(cc * 16, 16)] = jnp.zeros((16,), F32)

        @pl.loop(0, RPT // SC_C)
        def _(zi):
            pltpu.sync_copy(zbuf, acc.at[pl.ds(sid * RPT + zi * SC_C, SC_C)])

        plsc.subcore_barrier()

        @pl.loop(0, nchunk)
        def _(ci):
            off = (cid * NT + sid) * EW + ci * SC_C
            pltpu.sync_copy(col_hbm.at[pl.ds(off, SC_C)], idx)
            pltpu.sync_copy(ones, acc.at[idx], add=True)

        plsc.subcore_barrier()
        pltpu.sync_copy(acc.at[pl.ds(sid * RPT, RPT)],
                        deg_hbm.at[cid, pl.ds(sid * RPT, RPT)])

    return k(col_p)


def _edge_embed_body(ea_ref, w_ref, b_ref, out_ref):
    acc = jnp.dot(ea_ref[...].astype(BF16), w_ref[...], preferred_element_type=F32)
    out_ref[...] = _lrelu(acc + b_ref[...])


def _node_embed_body(x_ref, batch_ref, wn_ref, bn_ref,
                     h_ref, hpk_ref, gcnt_ref, gcr_ref):
    i = pl.program_id(0)
    h = _lrelu(jnp.dot(x_ref[...].astype(BF16), wn_ref[...],
                       preferred_element_type=F32) + bn_ref[...])
    h_ref[...] = h
    hpk_ref[...] = _pack(h)
    b = batch_ref[0, 0, :]
    ng = gcnt_ref.shape[0]
    ohT = (jax.lax.broadcasted_iota(I32, (ng, b.shape[0]), 0)
           == b[None, :]).astype(F32)
    cnt = jnp.sum(ohT, axis=1, keepdims=True)
    ohb = (b[:, None] == jax.lax.broadcasted_iota(
        I32, (b.shape[0], ng), 1)).astype(BF16)
    gcr = jnp.dot(jnp.ones((1, b.shape[0]), BF16), ohb,
                  preferred_element_type=F32)

    @pl.when(i == 0)
    def _():
        gcnt_ref[...] = jnp.zeros_like(gcnt_ref)
        gcr_ref[...] = jnp.zeros_like(gcr_ref)

    gcnt_ref[...] += jnp.broadcast_to(cnt, gcnt_ref.shape)
    gcr_ref[...] += gcr


def _edge_body(e_ref, hr_ref, hc_ref, row_ref, gcr_ref, gvec_ref,
               w1a_ref, w1b_ref, w1c_ref, b1_ref, w2_ref, b2_ref,
               w11a_ref, w11b_ref, b11_ref, eout_ref, m1_ref):
    e = e_ref[...]
    hr = _unpack_bf16(hr_ref[...])
    hc = _unpack_bf16(hc_ref[...])
    acc = jnp.dot(hr, w1a_ref[...], preferred_element_type=F32)
    acc = acc + jnp.dot(hc, w1b_ref[...], preferred_element_type=F32)
    acc = acc + jnp.dot(e.astype(BF16), w1c_ref[...], preferred_element_type=F32)
    ng = gvec_ref.shape[0]
    gcr = gcr_ref[...]
    tri = (jax.lax.broadcasted_iota(I32, (ng, ng), 0)
           < jax.lax.broadcasted_iota(I32, (ng, ng), 1)).astype(F32)
    starts = jnp.dot(gcr, tri, preferred_element_type=F32)
    ends = starts + gcr
    rowf = row_ref[0, 0, :].astype(F32)[:, None]
    oh = ((rowf >= starts) & (rowf < ends)).astype(BF16)
    acc = acc + jnp.dot(oh, gvec_ref[...], preferred_element_type=F32)
    hidden = jnp.maximum(acc + b1_ref[...], 0.0)
    e_new = jnp.dot(hidden.astype(BF16), w2_ref[...],
                    preferred_element_type=F32) + b2_ref[...]
    eout_ref[...] = e + e_new
    m1 = jnp.dot(e_new.astype(BF16), w11b_ref[...], preferred_element_type=F32)
    m1 = m1 + jnp.dot(hr, w11a_ref[...], preferred_element_type=F32) + b11_ref[...]
    m1_ref[...] = jnp.maximum(m1, 0.0)


def _node_body(h_ref, seg0_ref, seg1_ref, deg_ref, batch_ref, gb2_ref,
               w12_ref, b12_ref,
               w21a_ref, w21b_ref, b21_ref, w22_ref, b22_ref,
               hout_ref, hpk_ref, gsum_ref):
    i = pl.program_id(0)
    h = h_ref[...]
    dg = deg_ref[...]
    deg = dg[0, :, :1] + dg[1, :, :1]
    inv = 1.0 / jnp.maximum(deg, 1.0)
    nz = (deg > 0).astype(F32)
    aggbase = (seg0_ref[...] + seg1_ref[...]) * inv
    aggm = jnp.dot(aggbase.astype(BF16), w12_ref[...],
                   preferred_element_type=F32) + b12_ref[...] * nz
    b = batch_ref[0, 0, :]
    ng = gb2_ref.shape[0]
    oh = (b[:, None] == jax.lax.broadcasted_iota(
        I32, (b.shape[0], ng), 1)).astype(BF16)
    t = jnp.dot(h.astype(BF16), w21a_ref[...], preferred_element_type=F32)
    t = t + jnp.dot(aggm.astype(BF16), w21b_ref[...], preferred_element_type=F32)
    t = t + jnp.dot(oh, gb2_ref[...], preferred_element_type=F32)
    t = jnp.maximum(t + b21_ref[...], 0.0)
    h_new = jnp.dot(t.astype(BF16), w22_ref[...],
                    preferred_element_type=F32) + b22_ref[...]
    hout = h + h_new
    hout_ref[...] = hout
    hpk_ref[...] = _pack(hout)
    ohT = (jax.lax.broadcasted_iota(I32, (ng, b.shape[0]), 0)
           == b[None, :]).astype(BF16)
    gs = jnp.dot(ohT, h_new.astype(BF16), preferred_element_type=F32)

    @pl.when(i == 0)
    def _():
        gsum_ref[...] = jnp.zeros_like(gsum_ref)

    gsum_ref[...] += gs


def _glob_init_body(u_ref, wg_ref, bg_ref, w1d_ref, w21c_ref,
                    g_ref, gvec_ref, gb2_ref):
    g = _lrelu(jnp.dot(u_ref[...], wg_ref[...],
                       preferred_element_type=F32) + bg_ref[...])
    g_ref[...] = g
    gvec_ref[...] = jnp.dot(g, w1d_ref[...], preferred_element_type=F32).astype(BF16)
    gb2_ref[...] = jnp.dot(g, w21c_ref[...], preferred_element_type=F32).astype(BF16)


def _glob_body(g_ref, gsum_ref, gcnt_ref, wg1a_ref, wg1b_ref, bg1_ref,
               wg2_ref, bg2_ref, w1d_ref, w21c_ref, v1w_ref, v1b_ref,
               v2w_ref, v2b_ref, gout_ref, gvec_ref, gb2_ref, val_ref):
    g = g_ref[...]
    cnt = jnp.maximum(gcnt_ref[:, :1], 1.0)
    hmean = gsum_ref[...] / cnt
    mid = jnp.maximum(
        jnp.dot(g, wg1a_ref[...], preferred_element_type=F32)
        + jnp.dot(hmean, wg1b_ref[...], preferred_element_type=F32)
        + bg1_ref[...], 0.0)
    g_new = jnp.dot(mid, wg2_ref[...], preferred_element_type=F32) + bg2_ref[...]
    gout = g + g_new
    gout_ref[...] = gout
    gvec_ref[...] = jnp.dot(gout, w1d_ref[...],
                            preferred_element_type=F32).astype(BF16)
    gb2_ref[...] = jnp.dot(gout, w21c_ref[...],
                           preferred_element_type=F32).astype(BF16)
    v = _lrelu(jnp.dot(gout, v1w_ref[...], preferred_element_type=F32) + v1b_ref[...])
    val_ref[...] = jnp.dot(v, v2w_ref[...], preferred_element_type=F32) + v2b_ref[...]


def _decode_body(h_ref, dw_ref, db_ref, out_ref):
    out_ref[...] = jnp.dot(h_ref[...], dw_ref[...],
                           preferred_element_type=F32) + db_ref[...]


def _full(shape):
    n = len(shape)
    return pl.BlockSpec(shape, lambda i: (0,) * n)


def kernel(x, edge_index, edge_attr, u, batch, params):
    N = x.shape[0]
    E = edge_index.shape[1]
    G = u.shape[0]
    D = params["node_emb"]["w"].shape[1]
    H = params["passes"][0]["edge_mlp1"]["w"].shape[1]
    VD = params["value1"]["w"].shape[1]

    EH = E // 2
    EPH = _cdiv(EH, NW * SC_C) * (NW * SC_C)
    EP = 2 * EPH
    NP = _cdiv(N, NBK) * NBK
    if NP == N and EPH > EH:
        NP += NBK
    NEB = EPH // EB
    NNB = NP // NBK

    row = edge_index[0].astype(I32)
    col = edge_index[1].astype(I32)

    def pad_half(a, lo, hi, fill):
        seg = a[lo:hi]
        return jnp.concatenate(
            [seg, jnp.full((EPH - (hi - lo),), fill, I32)])

    row_h = [pad_half(row, 0, EH, 0), pad_half(row, EH, E, 0)]
    col_h = [pad_half(col, 0, EH, NP - 1), pad_half(col, EH, E, NP - 1)]
    col_p = jnp.concatenate(col_h)
    batch_i = batch.astype(I32)
    batch_p = jnp.concatenate([batch_i, jnp.full((NP - N,), G, I32)])
    batch3 = batch_p.reshape(NNB, 1, NBK)
    row3_h = [r.reshape(NEB, 1, EB) for r in row_h]

    x_p = jnp.pad(x, ((0, NP - N), (0, 0)))
    ea_h = [jnp.pad(edge_attr[:EH], ((0, EPH - EH), (0, 0))),
            jnp.pad(edge_attr[EH:], ((0, EPH - EH), (0, 0)))]

    def bias(lin):
        return lin["b"][None, :]

    packs = []
    for p in params["passes"]:
        W1 = p["edge_mlp1"]["w"]
        W11 = p["node_mlp11"]["w"]
        W21 = p["node_mlp21"]["w"]
        Wg1 = p["glob_mlp1"]["w"]
        packs.append(dict(
            w1a=W1[:D].astype(BF16), w1b=W1[D:2 * D].astype(BF16),
            w1c=W1[2 * D:3 * D].astype(BF16), w1d=W1[3 * D:],
            b1=bias(p["edge_mlp1"]),
            w2=p["edge_mlp2"]["w"].astype(BF16), b2=bias(p["edge_mlp2"]),
            w11a=W11[:D].astype(BF16), w11b=W11[D:].astype(BF16),
            b11=bias(p["node_mlp11"]),
            w12=p["node_mlp12"]["w"].astype(BF16), b12=bias(p["node_mlp12"]),
            w21a=W21[:D].astype(BF16), w21b=W21[D:D + H].astype(BF16),
            w21c=W21[D + H:], b21=bias(p["node_mlp21"]),
            w22=p["node_mlp22"]["w"].astype(BF16), b22=bias(p["node_mlp22"]),
            wg1a=Wg1[:D], wg1b=Wg1[D:], bg1=bias(p["glob_mlp1"]),
            wg2=p["glob_mlp2"]["w"], bg2=bias(p["glob_mlp2"]),
        ))

    egrid = (NEB,)
    ngrid = (NNB,)

    e_h = [pl.pallas_call(
        _edge_embed_body,
        grid=egrid,
        in_specs=[pl.BlockSpec((EB, edge_attr.shape[1]), lambda i: (i, 0)),
                  _full((edge_attr.shape[1], D)), _full((1, D))],
        out_specs=pl.BlockSpec((EB, D), lambda i: (i, 0)),
        out_shape=jax.ShapeDtypeStruct((EPH, D), F32),
    )(ea, params["edge_emb"]["w"].astype(BF16), bias(params["edge_emb"]))
        for ea in ea_h]

    h, hpk, gcnt, gcr = pl.pallas_call(
        _node_embed_body,
        grid=ngrid,
        in_specs=[pl.BlockSpec((NBK, x.shape[1]), lambda i: (i, 0)),
                  pl.BlockSpec((1, 1, NBK), lambda i: (i, 0, 0)),
                  _full((x.shape[1], D)), _full((1, D))],
        out_specs=[pl.BlockSpec((NBK, D), lambda i: (i, 0)),
                   pl.BlockSpec((NBK, D // 2), lambda i: (i, 0)),
                   pl.BlockSpec((G, 128), lambda i: (0, 0)),
                   pl.BlockSpec((1, G), lambda i: (0, 0))],
        out_shape=[jax.ShapeDtypeStruct((NP, D), F32),
                   jax.ShapeDtypeStruct((NP, D // 2), I32),
                   jax.ShapeDtypeStruct((G, 128), F32),
                   jax.ShapeDtypeStruct((1, G), F32)],
    )(x_p, batch3, params["node_emb"]["w"].astype(BF16),
      bias(params["node_emb"]))

    g, gvec, gb2 = pl.pallas_call(
        _glob_init_body,
        grid=(1,),
        in_specs=[_full((G, u.shape[1])), _full((u.shape[1], D)), _full((1, D)),
                  _full((D, H)), _full((D, H))],
        out_specs=[_full((G, D)), _full((G, H)), _full((G, H))],
        out_shape=[jax.ShapeDtypeStruct((G, D), F32),
                   jax.ShapeDtypeStruct((G, H), BF16),
                   jax.ShapeDtypeStruct((G, H), BF16)],
    )(u, params["glob_emb"]["w"], bias(params["glob_emb"]),
      packs[0]["w1d"], packs[0]["w21c"])

    deg2 = _sc_deg(col_p, NP)

    npass = len(params["passes"])
    for pi, pk in enumerate(packs):
        nxt = packs[pi + 1] if pi + 1 < npass else packs[0]

        seg_h = [None, None]
        for hf in range(2):
            hr, hc = _sc_gather(hpk, row_h[hf], col_h[hf])

            e_h[hf], m1 = pl.pallas_call(
                _edge_body,
                grid=egrid,
                in_specs=[pl.BlockSpec((EB, D), lambda i: (i, 0)),
                          pl.BlockSpec((EB, D // 2), lambda i: (i, 0)),
                          pl.BlockSpec((EB, D // 2), lambda i: (i, 0)),
                          pl.BlockSpec((1, 1, EB), lambda i: (i, 0, 0)),
                          _full((1, G)), _full((G, H)),
                          _full((D, H)), _full((D, H)), _full((D, H)),
                          _full((1, H)),
                          _full((H, D)), _full((1, D)),
                          _full((D, H)), _full((D, H)), _full((1, H))],
                out_specs=[pl.BlockSpec((EB, D), lambda i: (i, 0)),
                           pl.BlockSpec((EB, H), lambda i: (i, 0))],
                out_shape=[jax.ShapeDtypeStruct((EPH, D), F32),
                           jax.ShapeDtypeStruct((EPH, H), F32)],
            )(e_h[hf], hr, hc, row3_h[hf], gcr, gvec, pk["w1a"], pk["w1b"],
              pk["w1c"], pk["b1"], pk["w2"], pk["b2"], pk["w11a"],
              pk["w11b"], pk["b11"])

            seg_h[hf] = _sc_scatter(m1, col_h[hf], NP)

        h, hpk, gsum = pl.pallas_call(
            _node_body,
            grid=ngrid,
            in_specs=[pl.BlockSpec((NBK, D), lambda i: (i, 0)),
                      pl.BlockSpec((NBK, H), lambda i: (i, 0)),
                      pl.BlockSpec((NBK, H), lambda i: (i, 0)),
                      pl.BlockSpec((2, NBK, 128), lambda i: (0, i, 0)),
                      pl.BlockSpec((1, 1, NBK), lambda i: (i, 0, 0)),
                      _full((G, H)), _full((H, H)), _full((1, H)),
                      _full((D, H)), _full((H, H)), _full((1, H)),
                      _full((H, D)), _full((1, D))],
            out_specs=[pl.BlockSpec((NBK, D), lambda i: (i, 0)),
                       pl.BlockSpec((NBK, D // 2), lambda i: (i, 0)),
                       pl.BlockSpec((G, D), lambda i: (0, 0))],
            out_shape=[jax.ShapeDtypeStruct((NP, D), F32),
                       jax.ShapeDtypeStruct((NP, D // 2), I32),
                       jax.ShapeDtypeStruct((G, D), F32)],
        )(h, seg_h[0], seg_h[1], deg2, batch3, gb2, pk["w12"], pk["b12"],
          pk["w21a"], pk["w21b"], pk["b21"], pk["w22"], pk["b22"])

        g, gvec, gb2, val = pl.pallas_call(
            _glob_body,
            grid=(1,),
            in_specs=[_full((G, D)), _full((G, D)), _full((G, 128)),
                      _full((D, H)), _full((D, H)), _full((1, H)),
                      _full((H, D)), _full((1, D)),
                      _full((D, H)), _full((D, H)),
                      _full((D, VD)), _full((1, VD)),
                      _full((VD, 128)), _full((1, 128))],
            out_specs=[_full((G, D)), _full((G, H)), _full((G, H)),
                       _full((G, 128))],
            out_shape=[jax.ShapeDtypeStruct((G, D), F32),
                       jax.ShapeDtypeStruct((G, H), BF16),
                       jax.ShapeDtypeStruct((G, H), BF16),
                       jax.ShapeDtypeStruct((G, 128), F32)],
        )(g, gsum, gcnt, pk["wg1a"], pk["wg1b"], pk["bg1"], pk["wg2"],
          pk["bg2"], nxt["w1d"], nxt["w21c"], params["value1"]["w"],
          bias(params["value1"]),
          jnp.pad(params["value2"]["w"], ((0, 0), (0, 127))),
          jnp.pad(bias(params["value2"]), ((0, 0), (0, 127))))

    x_out = pl.pallas_call(
        _decode_body,
        grid=ngrid,
        in_specs=[pl.BlockSpec((NBK, D), lambda i: (i, 0)),
                  _full((D, 128)), _full((1, 128))],
        out_specs=pl.BlockSpec((NBK, 128), lambda i: (i, 0)),
        out_shape=jax.ShapeDtypeStruct((NP, 128), F32),
    )(h, jnp.pad(params["node_dec"]["w"], ((0, 0), (0, 127))),
      jnp.pad(bias(params["node_dec"]), ((0, 0), (0, 127))))

    return (x_out[:N, :1], val[:, :1])
